# Initial kernel scaffold; baseline (speedup 1.0000x reference)
import functools

import jax
import jax.numpy as jnp
from jax import lax
from jax.experimental import pallas as pl
from jax.experimental.pallas import tpu as pltpu
from jax.experimental.pallas import tpu_sc as plsc


def _make_spmm(N, D, E, NC, NS):
    W_ = NC * NS
    EW = E // W_
    K = 80
    C = EW // K
    RPT = N // NS
    mesh = plsc.VectorSubcoreMesh(core_axis_name="c", subcore_axis_name="s")

    @functools.partial(
        pl.kernel,
        out_type=jax.ShapeDtypeStruct((NC * N, D), jnp.float32),
        mesh=mesh,
        scratch_types=[
            pltpu.VMEM((K,), jnp.int32),
            pltpu.VMEM((K,), jnp.int32),
            pltpu.VMEM((K,), jnp.float32),
            pltpu.VMEM((K, D), jnp.float32),
            pltpu.VMEM_SHARED((N, D), jnp.float32),
            pltpu.SemaphoreType.DMA,
        ],
    )
    def spmm(x_hbm, src_hbm, dst_hbm, ev_hbm, z_hbm, out_hbm,
             src_v, dst_v, ev_v, rows_v, acc, sem):
        c = lax.axis_index("c")
        s = lax.axis_index("s")
        wid = c * NS + s

        pltpu.sync_copy(z_hbm, acc.at[pl.ds(s * RPT, RPT)])
        plsc.subcore_barrier()

        ebase = wid * EW

        def chunk(i, carry):
            base = pl.multiple_of(ebase + i * K, 8)
            pltpu.sync_copy(src_hbm.at[pl.ds(base, K)], src_v)
            pltpu.sync_copy(dst_hbm.at[pl.ds(base, K)], dst_v)
            pltpu.sync_copy(ev_hbm.at[pl.ds(base, K)], ev_v)
            pltpu.async_copy(x_hbm.at[src_v], rows_v, sem).wait()

            def srow(r, _):
                sc = ev_v[r]
                for j in range(D // 16):
                    sl = pl.ds(j * 16, 16)
                    rows_v[r, sl] = rows_v[r, sl] * sc
                return 0

            lax.fori_loop(0, K, srow, 0)
            pltpu.sync_copy(rows_v, acc.at[dst_v], add=True)
            return carry

        lax.fori_loop(0, C, chunk, 0)

        plsc.subcore_barrier()
        pltpu.sync_copy(acc.at[pl.ds(s * RPT, RPT)],
                        out_hbm.at[pl.ds(c * N + s * RPT, RPT)])

    return spmm


def _mm_body(p_ref, w_ref, o_ref):
    a = jnp.sum(p_ref[...], axis=0)
    o_ref[...] = jnp.dot(a, w_ref[...], preferred_element_type=jnp.float32)


def _matmul_partials(p, w):
    NC, N, D = p.shape
    DO = w.shape[1]
    BLK = 1000
    return pl.pallas_call(
        _mm_body,
        grid=(N // BLK,),
        in_specs=[pl.BlockSpec((NC, BLK, D), lambda i: (0, i, 0)),
                  pl.BlockSpec((D, DO), lambda i: (0, 0))],
        out_specs=pl.BlockSpec((BLK, DO), lambda i: (i, 0)),
        out_shape=jax.ShapeDtypeStruct((N, DO), jnp.float32),
    )(p, w)


def kernel(x, edge_index, edge_value, W):
    N, D = x.shape
    E = edge_value.shape[0]
    info = plsc.get_sparse_core_info()
    NC, NS = info.num_cores, info.num_subcores
    src = edge_index[0]
    dst = edge_index[1]
    z = jnp.zeros((N // NS, D), jnp.float32)
    spmm = _make_spmm(N, D, E, NC, NS)
    partials = spmm(x, src, dst, edge_value, z)
    return _matmul_partials(partials.reshape(NC, N, D), W)


# speedup vs baseline: 2.0126x; 2.0126x over previous
import functools

import jax
import jax.numpy as jnp
from jax import lax
from jax.experimental import pallas as pl
from jax.experimental.pallas import tpu as pltpu
from jax.experimental.pallas import tpu_sc as plsc


def _make_spmm(N, D, E, NC, NS):
    W_ = NC * NS
    NP = -(-N // (W_ * 8)) * (W_ * 8)
    SPT = NP // W_
    SEG = min(4000, E)
    NSEG = E // SEG
    KG = 128
    LCAP = SEG + KG + 16
    assert SEG % 16 == 0 and E % SEG == 0
    mesh = plsc.VectorSubcoreMesh(core_axis_name="c", subcore_axis_name="s",
                                  num_cores=NC, num_subcores=NS)

    @functools.partial(
        pl.kernel,
        out_type=jax.ShapeDtypeStruct((NP, D), jnp.float32),
        mesh=mesh,
        compiler_params=pltpu.CompilerParams(needs_layout_passes=False),
        scratch_types=[
            pltpu.VMEM((SEG,), jnp.int32),
            pltpu.VMEM((LCAP,), jnp.int32),
            pltpu.VMEM((LCAP,), jnp.int32),
            pltpu.VMEM((KG,), jnp.int32),
            pltpu.VMEM((KG,), jnp.float32),
            pltpu.VMEM((KG, D), jnp.float32),
            pltpu.VMEM((SPT + 8, D), jnp.float32),
            pltpu.SemaphoreType.DMA,
        ],
    )
    def spmm(x_hbm, src_hbm, dst_hbm, ev_hbm, z_hbm, out_hbm,
             dseg, elist, dlist, srcg, evg, rows, acc, sem):
        c = lax.axis_index("c")
        s = lax.axis_index("s")
        wid = c * NS + s
        base = wid * SPT

        pltpu.sync_copy(z_hbm, acc)

        iota16 = lax.iota(jnp.int32, 16)

        def process_chunk(off):
            off = pl.multiple_of(off, 8)
            pltpu.async_copy(src_hbm.at[elist.at[pl.ds(off, KG)]], srcg,
                             sem).wait()
            pltpu.async_copy(ev_hbm.at[elist.at[pl.ds(off, KG)]], evg,
                             sem).wait()
            pltpu.async_copy(x_hbm.at[srcg], rows, sem).wait()

            def accg(g, _):
                evv = evg[pl.ds(g * 16, 16)]
                dlv = dlist[pl.ds(off + g * 16, 16)]
                for r in range(16):
                    sc = evv[r]
                    dl = dlv[r]
                    row = g * 16 + r
                    for j in range(D // 16):
                        sl = pl.ds(j * 16, 16)
                        plsc.addupdate(acc.at[dl, sl], rows[row, sl] * sc)
                return 0

            lax.fori_loop(0, KG // 16, accg, 0)

        def seg_body(g, fill):
            segbase = g * SEG
            pltpu.sync_copy(dst_hbm.at[pl.ds(segbase, SEG)], dseg)

            def scan(v, cnt):
                dv = dseg[pl.ds(v * 16, 16)]
                lv = dv - base
                mask = (lv >= 0) & (lv < SPT)
                eids = iota16 + (segbase + v * 16)
                plsc.store_compressed(elist.at[pl.ds(cnt, 16)], eids,
                                      mask=mask)
                plsc.store_compressed(dlist.at[pl.ds(cnt, 16)], lv,
                                      mask=mask)
                return cnt + plsc.all_reduce_population_count(mask)[0]

            fill = lax.fori_loop(0, SEG // 16, scan, fill)

            def drain(k, off):
                @pl.when(fill - off >= KG)
                def _():
                    process_chunk(off)
                return jnp.where(fill - off >= KG, off + KG, off)

            off0 = lax.fori_loop(0, SEG // KG + 1, drain, jnp.int32(0))

            rem = fill - off0

            def mv(k, _):
                @pl.when(k * 16 < rem)
                def _():
                    sl_from = pl.ds(off0 + k * 16, 16)
                    sl_to = pl.ds(k * 16, 16)
                    elist[sl_to] = elist[sl_from]
                    dlist[sl_to] = dlist[sl_from]
                return 0

            lax.fori_loop(0, KG // 16, mv, 0)
            return rem

        fill = lax.fori_loop(0, NSEG, seg_body, jnp.int32(0))

        for k in range(KG // 16):
            elist[pl.ds(fill + k * 16, 16)] = jnp.zeros((16,), jnp.int32)
            dlist[pl.ds(fill + k * 16, 16)] = jnp.full((16,), SPT, jnp.int32)

        @pl.when(fill > 0)
        def _():
            process_chunk(jnp.int32(0))

        pltpu.sync_copy(acc.at[pl.ds(0, SPT)], out_hbm.at[pl.ds(base, SPT)])

    return spmm


def _mm_body(p_ref, w_ref, o_ref):
    o_ref[...] = jnp.dot(p_ref[...], w_ref[...],
                         preferred_element_type=jnp.float32)


def _matmul(p, w):
    NPD, D = p.shape
    DO = w.shape[1]
    BLK = 1024
    return pl.pallas_call(
        _mm_body,
        grid=(NPD // BLK,),
        in_specs=[pl.BlockSpec((BLK, D), lambda i: (i, 0)),
                  pl.BlockSpec((D, DO), lambda i: (0, 0))],
        out_specs=pl.BlockSpec((BLK, DO), lambda i: (i, 0)),
        out_shape=jax.ShapeDtypeStruct((NPD, DO), jnp.float32),
    )(p, w)


def kernel(x, edge_index, edge_value, W):
    N, D = x.shape
    E = edge_value.shape[0]
    info = plsc.get_sparse_core_info()
    NC, NS = info.num_cores, info.num_subcores
    src = edge_index[0]
    dst = edge_index[1]
    W_ = NC * NS
    NP = -(-N // (W_ * 8)) * (W_ * 8)
    SPT = NP // W_
    z = jnp.zeros((SPT + 8, D), jnp.float32)
    spmm = _make_spmm(N, D, E, NC, NS)
    agg = spmm(x, src, dst, edge_value, z)
    return _matmul(agg, W)[:N]
